# baseline (speedup 1.0000x reference)
import functools

import jax
import jax.numpy as jnp
from jax import lax
from jax.experimental import pallas as pl
from jax.experimental.pallas import tpu as pltpu
from jax.experimental.pallas import tpu_sc as plsc

_D = 128
_NW = 32
_NS = 16
_C = 112
_JPW = 28
_S = _C * _JPW
_TSLICE = 632
_KH = 6


def _unpool_body(b, v, table_hbm, idx_hbm, out_hbm,
                 table_sp, idx_v, rows0, rows1,
                 gsem0, gsem1, ssem0, ssem1, tsem):
    sid = lax.axis_index("s")
    wid = sid * 2 + lax.axis_index("c")
    base_w = jnp.minimum(wid * _S, b - _S)

    rows = (rows0, rows1)
    gsem = (gsem0, gsem1)
    ssem = (ssem0, ssem1)

    base_t = jnp.minimum(sid * _TSLICE, v - _TSLICE)
    pltpu.async_copy(
        table_hbm.at[pl.ds(base_t, _TSLICE)],
        table_sp.at[pl.ds(base_t, _TSLICE)], tsem)
    pltpu.sync_copy(idx_hbm.at[pl.ds(base_w, _S)], idx_v)

    def gather_h(j, p):
        pltpu.async_copy(
            table_hbm.at[idx_v.at[pl.ds(j * _C, _C)]], rows[p], gsem[p])

    def gather_h_wait(j, p):
        pltpu.make_async_copy(
            table_hbm.at[idx_v.at[pl.ds(j * _C, _C)]], rows[p], gsem[p]
        ).wait()

    def gather(j, p):
        pltpu.async_copy(
            table_sp.at[idx_v.at[pl.ds(j * _C, _C)]], rows[p], gsem[p])

    def gather_wait(j, p):
        pltpu.make_async_copy(
            table_sp.at[idx_v.at[pl.ds(j * _C, _C)]], rows[p], gsem[p]
        ).wait()

    def scatter_start(j, p):
        pltpu.async_copy(
            rows[p], out_hbm.at[pl.ds(base_w + j * _C, _C)], ssem[p])

    def scatter_wait(j, p):
        pltpu.make_async_copy(
            rows[p], out_hbm.at[pl.ds(base_w + j * _C, _C)], ssem[p]).wait()

    gather_h(0, 0)
    for j in range(_KH):
        if j >= 1:
            scatter_wait(j - 1, (j - 1) % 2)
        nx = j + 1
        if nx < _KH:
            gather_h(nx, nx % 2)
        elif nx == _KH:
            pltpu.make_async_copy(
                table_hbm.at[pl.ds(base_t, _TSLICE)],
                table_sp.at[pl.ds(base_t, _TSLICE)], tsem).wait()
            plsc.subcore_barrier()
            gather(nx, nx % 2)
        gather_h_wait(j, j % 2)
        scatter_start(j, j % 2)

    def body(jj, carry):
        for t in range(2):
            j = jj * 2 + t

            scatter_wait(j - 1, 1 - t)

            @pl.when(j + 1 < _JPW)
            def _(j=j, t=t):
                gather(j + 1, 1 - t)

            gather_wait(j, t)
            scatter_start(j, t)
        return carry

    lax.fori_loop(_KH // 2, _JPW // 2, body, 0)
    scatter_wait(_JPW - 1, (_JPW - 1) % 2)


def kernel(x_pooled, batch, num_nodes):
    del num_nodes
    b = batch.shape[0]
    v = x_pooled.shape[0]
    idx = batch.astype(jnp.int32)

    f = pl.kernel(
        functools.partial(_unpool_body, b, v),
        mesh=plsc.VectorSubcoreMesh(core_axis_name="c", subcore_axis_name="s"),
        out_type=jax.ShapeDtypeStruct((b, _D), jnp.float32),
        scratch_types=[
            pltpu.VMEM_SHARED((10000, _D), jnp.float32),
            pltpu.VMEM((_S,), jnp.int32),
            pltpu.VMEM((_C, _D), jnp.float32),
            pltpu.VMEM((_C, _D), jnp.float32),
            pltpu.SemaphoreType.DMA,
            pltpu.SemaphoreType.DMA,
            pltpu.SemaphoreType.DMA,
            pltpu.SemaphoreType.DMA,
            pltpu.SemaphoreType.DMA,
        ],
    )
    return f(x_pooled, idx)


# speedup vs baseline: 1.1534x; 1.1534x over previous
import functools

import jax
import jax.numpy as jnp
from jax import lax
from jax.experimental import pallas as pl
from jax.experimental.pallas import tpu as pltpu
from jax.experimental.pallas import tpu_sc as plsc

_D = 128
_NW = 32
_NS = 16
_C = 112
_JPW = 28
_S = _C * _JPW
_TSLICE = 632
_KH = 2


def _unpool_body(b, v, table_hbm, idx_hbm, out_hbm,
                 table_sp, idx_v, rows0, rows1,
                 gsem0, gsem1, ssem0, ssem1, tsem):
    sid = lax.axis_index("s")
    wid = sid * 2 + lax.axis_index("c")
    base_w = jnp.minimum(wid * _S, b - _S)

    rows = (rows0, rows1)
    gsem = (gsem0, gsem1)
    ssem = (ssem0, ssem1)

    base_t = jnp.minimum(sid * _TSLICE, v - _TSLICE)
    pltpu.async_copy(
        table_hbm.at[pl.ds(base_t, _TSLICE)],
        table_sp.at[pl.ds(base_t, _TSLICE)], tsem)
    pltpu.sync_copy(idx_hbm.at[pl.ds(base_w, _S)], idx_v)

    def gather_h(j, p):
        pltpu.async_copy(
            table_hbm.at[idx_v.at[pl.ds(j * _C, _C)]], rows[p], gsem[p])

    def gather_h_wait(j, p):
        pltpu.make_async_copy(
            table_hbm.at[idx_v.at[pl.ds(j * _C, _C)]], rows[p], gsem[p]
        ).wait()

    def gather(j, p):
        pltpu.async_copy(
            table_sp.at[idx_v.at[pl.ds(j * _C, _C)]], rows[p], gsem[p])

    def gather_wait(j, p):
        pltpu.make_async_copy(
            table_sp.at[idx_v.at[pl.ds(j * _C, _C)]], rows[p], gsem[p]
        ).wait()

    def scatter_start(j, p):
        pltpu.async_copy(
            rows[p], out_hbm.at[pl.ds(base_w + j * _C, _C)], ssem[p])

    def scatter_wait(j, p):
        pltpu.make_async_copy(
            rows[p], out_hbm.at[pl.ds(base_w + j * _C, _C)], ssem[p]).wait()

    gather_h(0, 0)
    for j in range(_KH):
        if j >= 1:
            scatter_wait(j - 1, (j - 1) % 2)
        nx = j + 1
        if nx < _KH:
            gather_h(nx, nx % 2)
        elif nx == _KH:
            pltpu.make_async_copy(
                table_hbm.at[pl.ds(base_t, _TSLICE)],
                table_sp.at[pl.ds(base_t, _TSLICE)], tsem).wait()
            plsc.subcore_barrier()
            gather(nx, nx % 2)
        gather_h_wait(j, j % 2)
        scatter_start(j, j % 2)

    def body(jj, carry):
        for t in range(2):
            j = jj * 2 + t

            scatter_wait(j - 1, 1 - t)

            @pl.when(j + 1 < _JPW)
            def _(j=j, t=t):
                gather(j + 1, 1 - t)

            gather_wait(j, t)
            scatter_start(j, t)
        return carry

    lax.fori_loop(_KH // 2, _JPW // 2, body, 0)
    scatter_wait(_JPW - 1, (_JPW - 1) % 2)


def kernel(x_pooled, batch, num_nodes):
    del num_nodes
    b = batch.shape[0]
    v = x_pooled.shape[0]
    idx = batch.astype(jnp.int32)

    f = pl.kernel(
        functools.partial(_unpool_body, b, v),
        mesh=plsc.VectorSubcoreMesh(core_axis_name="c", subcore_axis_name="s"),
        out_type=jax.ShapeDtypeStruct((b, _D), jnp.float32),
        scratch_types=[
            pltpu.VMEM_SHARED((10000, _D), jnp.float32),
            pltpu.VMEM((_S,), jnp.int32),
            pltpu.VMEM((_C, _D), jnp.float32),
            pltpu.VMEM((_C, _D), jnp.float32),
            pltpu.SemaphoreType.DMA,
            pltpu.SemaphoreType.DMA,
            pltpu.SemaphoreType.DMA,
            pltpu.SemaphoreType.DMA,
            pltpu.SemaphoreType.DMA,
        ],
    )
    return f(x_pooled, idx)


# speedup vs baseline: 1.1582x; 1.0042x over previous
import functools

import jax
import jax.numpy as jnp
from jax import lax
from jax.experimental import pallas as pl
from jax.experimental.pallas import tpu as pltpu
from jax.experimental.pallas import tpu_sc as plsc

_D = 128
_NW = 32
_C = 112
_JPW = 28
_S = _C * _JPW
_TSLICE = 632
_KH = 2


def _unpool_body(b, v, table_hbm, idx_hbm, out_hbm,
                 table_sp, idx_v, rows2, gsem, ssem, tsem):
    sid = lax.axis_index("s")
    wid = sid * 2 + lax.axis_index("c")
    base_w = jnp.minimum(wid * _S, b - _S)

    base_t = jnp.minimum(sid * _TSLICE, v - _TSLICE)
    pltpu.async_copy(
        table_hbm.at[pl.ds(base_t, _TSLICE)],
        table_sp.at[pl.ds(base_t, _TSLICE)], tsem)
    pltpu.sync_copy(idx_hbm.at[pl.ds(base_w, _S)], idx_v)

    def bufs(j):
        p = j % 2
        return (rows2.at[pl.ds(p * _C, _C)], gsem.at[p], ssem.at[p])

    def gather_h(j):
        rows, g, _ = bufs(j)
        pltpu.async_copy(table_hbm.at[idx_v.at[pl.ds(j * _C, _C)]], rows, g)

    def gather_h_wait(j):
        rows, g, _ = bufs(j)
        pltpu.make_async_copy(
            table_hbm.at[idx_v.at[pl.ds(j * _C, _C)]], rows, g).wait()

    def gather(j):
        rows, g, _ = bufs(j)
        pltpu.async_copy(table_sp.at[idx_v.at[pl.ds(j * _C, _C)]], rows, g)

    def gather_wait(j):
        rows, g, _ = bufs(j)
        pltpu.make_async_copy(
            table_sp.at[idx_v.at[pl.ds(j * _C, _C)]], rows, g).wait()

    def scatter_start(j):
        rows, _, s = bufs(j)
        pltpu.async_copy(rows, out_hbm.at[pl.ds(base_w + j * _C, _C)], s)

    def scatter_wait(j):
        rows, _, s = bufs(j)
        pltpu.make_async_copy(
            rows, out_hbm.at[pl.ds(base_w + j * _C, _C)], s).wait()

    gather_h(0)

    def peel(j, carry):
        @pl.when(j >= 1)
        def _():
            scatter_wait(j - 1)

        @pl.when(j + 1 < _KH)
        def _():
            gather_h(j + 1)

        @pl.when(j + 1 == _KH)
        def _():
            pltpu.make_async_copy(
                table_hbm.at[pl.ds(base_t, _TSLICE)],
                table_sp.at[pl.ds(base_t, _TSLICE)], tsem).wait()
            plsc.subcore_barrier()
            gather(j + 1)

        gather_h_wait(j)
        scatter_start(j)
        return carry

    lax.fori_loop(0, _KH, peel, 0)

    def body(j, carry):
        scatter_wait(j - 1)

        @pl.when(j + 1 < _JPW)
        def _():
            gather(j + 1)

        gather_wait(j)
        scatter_start(j)
        return carry

    lax.fori_loop(_KH, _JPW, body, 0)
    scatter_wait(_JPW - 1)


def kernel(x_pooled, batch, num_nodes):
    del num_nodes
    b = batch.shape[0]
    v = x_pooled.shape[0]
    idx = batch.astype(jnp.int32)

    f = pl.kernel(
        functools.partial(_unpool_body, b, v),
        mesh=plsc.VectorSubcoreMesh(core_axis_name="c", subcore_axis_name="s"),
        out_type=jax.ShapeDtypeStruct((b, _D), jnp.float32),
        scratch_types=[
            pltpu.VMEM_SHARED((10000, _D), jnp.float32),
            pltpu.VMEM((_S,), jnp.int32),
            pltpu.VMEM((2 * _C, _D), jnp.float32),
            pltpu.SemaphoreType.DMA((2,)),
            pltpu.SemaphoreType.DMA((2,)),
            pltpu.SemaphoreType.DMA,
        ],
    )
    return f(x_pooled, idx)
